# baseline (speedup 1.0000x reference)
import functools

import jax
import jax.numpy as jnp
from jax import lax
from jax.experimental import pallas as pl
from jax.experimental.pallas import tpu as pltpu
from jax.experimental.pallas import tpu_sc as plsc

_RADIUS2 = 0.1 * 0.1
_NSAMPLE = 64.0
_MARGIN = 0.5
_N = 8192
_NC = 2
_NS = 16
_NW = _NC * _NS
_ROWS_PER_W = _N // _NW
_L = 16
_PAIRS = _ROWS_PER_W // (2 * _L)
_UNROLL = 16


def _bf16_round(v):
    u = lax.bitcast_convert_type(v, jnp.uint32)
    lsb = lax.shift_right_logical(u, jnp.uint32(16)) & jnp.uint32(1)
    u = (u + jnp.uint32(0x7FFF) + lsb) & jnp.uint32(0xFFFF0000)
    return lax.bitcast_convert_type(u, jnp.float32)


def _sc_body(x_hbm, y_hbm, z_hbm, s_hbm, out_hbm, xv, yv, zv, sv, hv, accv,
             cntv, smv, mxv):
    cid = lax.axis_index("c")
    sid = lax.axis_index("s")
    wid = sid * _NC + cid

    pltpu.sync_copy(x_hbm, xv)
    pltpu.sync_copy(y_hbm, yv)
    pltpu.sync_copy(z_hbm, zv)
    pltpu.sync_copy(s_hbm, sv)

    def prep(i, _):
        o = pl.multiple_of(i * _L, 8)
        x = xv[pl.ds(o, _L)]
        y = yv[pl.ds(o, _L)]
        z = zv[pl.ds(o, _L)]
        hv[pl.ds(o, _L)] = (x * x + y * y + z * z) * 0.5
        xv[pl.ds(o, _L)] = _bf16_round(x)
        yv[pl.ds(o, _L)] = _bf16_round(y)
        zv[pl.ds(o, _L)] = _bf16_round(z)
        return 0
    lax.fori_loop(0, _N // _L, prep, 0)

    zeros = jnp.zeros((_L,), jnp.float32)
    r2h = jnp.float32(_RADIUS2 * 0.5)

    def load_q(base):
        base = pl.multiple_of(base, 8)
        return (xv[pl.ds(base, _L)], yv[pl.ds(base, _L)],
                zv[pl.ds(base, _L)], hv[pl.ds(base, _L)] - r2h)

    def capped_rescan(q):
        qx, qy, qz, qh = q

        def colc(it, carry):
            cnt, sm, mx = carry
            j0 = pl.multiple_of(it * _UNROLL, 8)
            cx = xv[pl.ds(j0, _UNROLL)]
            cy = yv[pl.ds(j0, _UNROLL)]
            cz = zv[pl.ds(j0, _UNROLL)]
            ch = hv[pl.ds(j0, _UNROLL)]
            cs = sv[pl.ds(j0, _UNROLL)]
            for k in range(_UNROLL):
                dot = qx * cx[k] + qy * cy[k] + qz * cz[k]
                take = (dot > qh + ch[k]) & (cnt < _NSAMPLE)
                t = jnp.where(take, cs[k], 0.0)
                sm = sm + t
                mx = jnp.maximum(mx, t)
                cnt = cnt + jnp.where(take, 1.0, 0.0)
            return (cnt, sm, mx)

        return lax.fori_loop(0, _N // _UNROLL, colc, (zeros, zeros, zeros))

    def finish(q, cnt, sm, mx):
        cmax = cnt[0]
        for k in range(1, _L):
            cmax = jnp.maximum(cmax, cnt[k])
        cntv[...] = cnt
        smv[...] = sm
        mxv[...] = mx

        @pl.when(cmax > _NSAMPLE)
        def _rescan():
            c2, s2, m2 = capped_rescan(q)
            cntv[...] = c2
            smv[...] = s2
            mxv[...] = m2

        cnt, sm, mx = cntv[...], smv[...], mxv[...]
        mean = sm / jnp.maximum(cnt, 1.0)
        return jnp.maximum(mean - mx + _MARGIN, 0.0)

    def pair_body(p, acc):
        q0 = load_q(wid * _ROWS_PER_W + p * 2 * _L)
        q1 = load_q(wid * _ROWS_PER_W + p * 2 * _L + _L)
        qx0, qy0, qz0, qh0 = q0
        qx1, qy1, qz1, qh1 = q1

        def colf(it, carry):
            c0, s0, m0, c1, s1, m1 = carry
            j0 = pl.multiple_of(it * _UNROLL, 8)
            cx = xv[pl.ds(j0, _UNROLL)]
            cy = yv[pl.ds(j0, _UNROLL)]
            cz = zv[pl.ds(j0, _UNROLL)]
            ch = hv[pl.ds(j0, _UNROLL)]
            cs = sv[pl.ds(j0, _UNROLL)]
            for k in range(_UNROLL):
                bx, by, bz, hh, ss = cx[k], cy[k], cz[k], ch[k], cs[k]
                dot0 = qx0 * bx + qy0 * by + qz0 * bz
                in0 = dot0 > qh0 + hh
                t0 = jnp.where(in0, ss, 0.0)
                s0 = s0 + t0
                m0 = jnp.maximum(m0, t0)
                c0 = c0 + jnp.where(in0, 1.0, 0.0)
                dot1 = qx1 * bx + qy1 * by + qz1 * bz
                in1 = dot1 > qh1 + hh
                t1 = jnp.where(in1, ss, 0.0)
                s1 = s1 + t1
                m1 = jnp.maximum(m1, t1)
                c1 = c1 + jnp.where(in1, 1.0, 0.0)
            return (c0, s0, m0, c1, s1, m1)

        c0, s0, m0, c1, s1, m1 = lax.fori_loop(
            0, _N // _UNROLL, colf, (zeros,) * 6)
        return acc + finish(q0, c0, s0, m0) + finish(q1, c1, s1, m1)

    accv[...] = lax.fori_loop(0, _PAIRS, pair_body, zeros)
    pltpu.sync_copy(accv, out_hbm.at[wid])


@functools.partial(
    pl.kernel,
    mesh=plsc.VectorSubcoreMesh(core_axis_name="c", subcore_axis_name="s"),
    out_type=jax.ShapeDtypeStruct((_NW, _L), jnp.float32),
    scratch_types=[
        pltpu.VMEM((_N,), jnp.float32),
        pltpu.VMEM((_N,), jnp.float32),
        pltpu.VMEM((_N,), jnp.float32),
        pltpu.VMEM((_N,), jnp.float32),
        pltpu.VMEM((_N,), jnp.float32),
        pltpu.VMEM((_L,), jnp.float32),
        pltpu.VMEM((_L,), jnp.float32),
        pltpu.VMEM((_L,), jnp.float32),
        pltpu.VMEM((_L,), jnp.float32),
    ],
)
def _peakness_sc(x_hbm, y_hbm, z_hbm, s_hbm, out_hbm, xv, yv, zv, sv, hv,
                 accv, cntv, smv, mxv):
    _sc_body(x_hbm, y_hbm, z_hbm, s_hbm, out_hbm, xv, yv, zv, sv, hv, accv,
             cntv, smv, mxv)


@jax.jit
def kernel(xyz, scores):
    xt = jnp.transpose(xyz)
    x = xt[0] + jnp.float32(0.0)
    y = xt[1] + jnp.float32(0.0)
    z = xt[2] + jnp.float32(0.0)
    partial = _peakness_sc(x, y, z, scores)
    return jnp.sum(partial) / jnp.float32(_N)


# speedup vs baseline: 1.9807x; 1.9807x over previous
import functools

import jax
import jax.numpy as jnp
from jax import lax
from jax.experimental import pallas as pl
from jax.experimental.pallas import tpu as pltpu
from jax.experimental.pallas import tpu_sc as plsc

_RADIUS2 = 0.1 * 0.1
_NSAMPLE = 64.0
_MARGIN = 0.5
_N = 8192
_NC = 2
_NS = 16
_NW = _NC * _NS
_ROWS_PER_W = _N // _NW
_L = 16
_G = 4
_PAIRS = _ROWS_PER_W // (_G * _L)
_UNROLL = 16


def _bf16_round(v):
    u = lax.bitcast_convert_type(v, jnp.uint32)
    lsb = lax.shift_right_logical(u, jnp.uint32(16)) & jnp.uint32(1)
    u = (u + jnp.uint32(0x7FFF) + lsb) & jnp.uint32(0xFFFF0000)
    return lax.bitcast_convert_type(u, jnp.float32)


def _sc_body(x_hbm, y_hbm, z_hbm, s_hbm, out_hbm, xv, yv, zv, sv, hv, accv,
             cntv, smv, mxv, *acc_refs):
    sas = list(acc_refs[:_G])
    cas = list(acc_refs[_G:])
    cid = lax.axis_index("c")
    sid = lax.axis_index("s")
    wid = sid * _NC + cid

    pltpu.sync_copy(x_hbm, xv)
    pltpu.sync_copy(y_hbm, yv)
    pltpu.sync_copy(z_hbm, zv)
    pltpu.sync_copy(s_hbm, sv)

    def prep(i, _):
        o = pl.multiple_of(i * _L, 8)
        x = xv[pl.ds(o, _L)]
        y = yv[pl.ds(o, _L)]
        z = zv[pl.ds(o, _L)]
        hv[pl.ds(o, _L)] = (x * x + y * y + z * z) * 0.5
        xv[pl.ds(o, _L)] = _bf16_round(x)
        yv[pl.ds(o, _L)] = _bf16_round(y)
        zv[pl.ds(o, _L)] = _bf16_round(z)
        return 0
    lax.fori_loop(0, _N // _L, prep, 0)

    zeros = jnp.zeros((_L,), jnp.float32)
    r2h = jnp.float32(_RADIUS2 * 0.5)

    def load_q(base):
        base = pl.multiple_of(base, 8)
        return (xv[pl.ds(base, _L)], yv[pl.ds(base, _L)],
                zv[pl.ds(base, _L)], hv[pl.ds(base, _L)] - r2h)

    def capped_rescan(q):
        qx, qy, qz, qh = q

        def colc(it, carry):
            cnt, sm, mx = carry
            j0 = pl.multiple_of(it * _UNROLL, 8)
            cx = xv[pl.ds(j0, _UNROLL)]
            cy = yv[pl.ds(j0, _UNROLL)]
            cz = zv[pl.ds(j0, _UNROLL)]
            ch = hv[pl.ds(j0, _UNROLL)]
            cs = sv[pl.ds(j0, _UNROLL)]
            for k in range(_UNROLL):
                dot = qx * cx[k] + qy * cy[k] + qz * cz[k]
                take = (dot > qh + ch[k]) & (cnt < _NSAMPLE)
                t = jnp.where(take, cs[k], 0.0)
                sm = sm + t
                mx = jnp.maximum(mx, t)
                cnt = cnt + jnp.where(take, 1.0, 0.0)
            return (cnt, sm, mx)

        return lax.fori_loop(0, _N // _UNROLL, colc, (zeros, zeros, zeros))

    def finish(q, cnt, sm, mx):
        cmax = cnt[0]
        for k in range(1, _L):
            cmax = jnp.maximum(cmax, cnt[k])
        cntv[...] = cnt
        smv[...] = sm
        mxv[...] = mx

        @pl.when(cmax > _NSAMPLE)
        def _rescan():
            c2, s2, m2 = capped_rescan(q)
            cntv[...] = c2
            smv[...] = s2
            mxv[...] = m2

        cnt, sm, mx = cntv[...], smv[...], mxv[...]
        mean = sm / jnp.maximum(cnt, 1.0)
        return jnp.maximum(mean - mx + _MARGIN, 0.0)

    def pair_body(p, acc):
        qs = [load_q(wid * _ROWS_PER_W + (p * _G + g) * _L)
              for g in range(_G)]
        for g in range(_G):
            sas[g][...] = zeros
            cas[g][...] = zeros

        def colf(it, ms):
            j0 = pl.multiple_of(it * _UNROLL, 8)
            cx = xv[pl.ds(j0, _UNROLL)]
            cy = yv[pl.ds(j0, _UNROLL)]
            cz = zv[pl.ds(j0, _UNROLL)]
            ch = hv[pl.ds(j0, _UNROLL)]
            cs = sv[pl.ds(j0, _UNROLL)]
            ms = list(ms)
            for k in range(_UNROLL):
                bx, by, bz, hh, ss = cx[k], cy[k], cz[k], ch[k], cs[k]
                for g in range(_G):
                    qx, qy, qz, qh = qs[g]
                    dot = qx * bx + qy * by + qz * bz
                    ing = dot > qh + hh
                    t = jnp.where(ing, ss, 0.0)
                    plsc.addupdate(sas[g].at[...], t)
                    ms[g] = jnp.maximum(ms[g], t)
                    plsc.addupdate(cas[g].at[...], jnp.where(ing, 1.0, 0.0))
            return tuple(ms)

        ms = lax.fori_loop(0, _N // _UNROLL, colf, (zeros,) * _G)
        for g in range(_G):
            acc = acc + finish(qs[g], cas[g][...], sas[g][...], ms[g])
        return acc

    accv[...] = lax.fori_loop(0, _PAIRS, pair_body, zeros)
    pltpu.sync_copy(accv, out_hbm.at[wid])


@functools.partial(
    pl.kernel,
    mesh=plsc.VectorSubcoreMesh(core_axis_name="c", subcore_axis_name="s"),
    out_type=jax.ShapeDtypeStruct((_NW, _L), jnp.float32),
    scratch_types=[
        pltpu.VMEM((_N,), jnp.float32),
        pltpu.VMEM((_N,), jnp.float32),
        pltpu.VMEM((_N,), jnp.float32),
        pltpu.VMEM((_N,), jnp.float32),
        pltpu.VMEM((_N,), jnp.float32),
        pltpu.VMEM((_L,), jnp.float32),
        pltpu.VMEM((_L,), jnp.float32),
        pltpu.VMEM((_L,), jnp.float32),
        pltpu.VMEM((_L,), jnp.float32),
    ] + [pltpu.VMEM((_L,), jnp.float32)] * (2 * _G),
)
def _peakness_sc(x_hbm, y_hbm, z_hbm, s_hbm, out_hbm, xv, yv, zv, sv, hv,
                 accv, cntv, smv, mxv, *acc_refs):
    _sc_body(x_hbm, y_hbm, z_hbm, s_hbm, out_hbm, xv, yv, zv, sv, hv, accv,
             cntv, smv, mxv, *acc_refs)


@jax.jit
def kernel(xyz, scores):
    xt = jnp.transpose(xyz)
    x = xt[0] + jnp.float32(0.0)
    y = xt[1] + jnp.float32(0.0)
    z = xt[2] + jnp.float32(0.0)
    partial = _peakness_sc(x, y, z, scores)
    return jnp.sum(partial) / jnp.float32(_N)
